# Initial kernel scaffold; baseline (speedup 1.0000x reference)
import jax
import jax.numpy as jnp
from jax.experimental import pallas as pl


def kernel(x_user, x_item, edge_index_u2i, edge_index_i2u, W_0_u2i, as_0_u2i, ad_0_u2i, b_0_u2i, W_0_i2u, as_0_i2u, ad_0_i2u, b_0_i2u, W_1_u2i, as_1_u2i, ad_1_u2i, b_1_u2i, W_1_i2u, as_1_i2u, ad_1_i2u, b_1_i2u):
    raise NotImplementedError("write your pallas kernel here")


# speedup vs baseline: 21.6094x; 21.6094x over previous
import functools

import jax
import jax.numpy as jnp
from jax import lax
from jax.experimental import pallas as pl
from jax.experimental.pallas import tpu as pltpu
from jax.experimental.pallas import tpu_sc as plsc

N = 10000
H = 128
E = 320000
NS = 16
NC = 2
LANES = 16
EPT = E // NS
EC = 80
NCHUNK = EPT // EC
ROWB = 1000
DC = 40
EPS = 1e-16


_BLK = 1000


def _proj_body(xu_ref, xi_ref, wa_ref, asa_ref, ada_ref, wb_ref, asb_ref,
               adb_ref, ha_ref, saa_ref, daa_ref, hb_ref, sab_ref, dab_ref):
    xu = xu_ref[...]
    xi = xi_ref[...]
    wa = wa_ref[...]
    wb = wb_ref[...]
    ha = jnp.dot(xu, wa, preferred_element_type=jnp.float32)
    hb = jnp.dot(xi, wb, preferred_element_type=jnp.float32)
    ha_ref[...] = ha
    hb_ref[...] = hb
    saa_ref[...] = jnp.sum(ha * asa_ref[...], axis=1, keepdims=True)
    sab_ref[...] = jnp.sum(hb * asb_ref[...], axis=1, keepdims=True)
    va = jnp.sum(wa * ada_ref[...], axis=1, keepdims=True)
    vb = jnp.sum(wb * adb_ref[...], axis=1, keepdims=True)
    daa_ref[...] = jnp.dot(xi, va, preferred_element_type=jnp.float32)
    dab_ref[...] = jnp.dot(xu, vb, preferred_element_type=jnp.float32)


def _project(xu, xi, wa, asa, ada, wb, asb, adb):
    grid = (N // _BLK,)
    row = pl.BlockSpec((_BLK, H), lambda i: (i, 0))
    full = pl.BlockSpec((H, H), lambda i: (0, 0))
    vec = pl.BlockSpec((1, H), lambda i: (0, 0))
    col = pl.BlockSpec((_BLK, 1), lambda i: (i, 0))
    f32 = jnp.float32
    return pl.pallas_call(
        _proj_body,
        grid=grid,
        in_specs=[row, row, full, vec, vec, full, vec, vec],
        out_specs=[row, col, col, row, col, col],
        out_shape=[
            jax.ShapeDtypeStruct((N, H), f32),
            jax.ShapeDtypeStruct((N, 1), f32),
            jax.ShapeDtypeStruct((N, 1), f32),
            jax.ShapeDtypeStruct((N, H), f32),
            jax.ShapeDtypeStruct((N, 1), f32),
            jax.ShapeDtypeStruct((N, 1), f32),
        ],
    )(xu, xi, wa, asa.reshape(1, H), ada.reshape(1, H),
      wb, asb.reshape(1, H), adb.reshape(1, H))


def _fin_body(ni_ref, di_ref, bi_ref, nu_ref, du_ref, bu_ref,
              xi_ref, xu_ref):
    xi_ref[...] = jnp.maximum(
        ni_ref[...] / (di_ref[...] + EPS) + bi_ref[...], 0.0)
    xu_ref[...] = jnp.maximum(
        nu_ref[...] / (du_ref[...] + EPS) + bu_ref[...], 0.0)


def _finalize(num_i, den_i, b_i, num_u, den_u, b_u):
    grid = (N // _BLK,)
    row = pl.BlockSpec((_BLK, H), lambda i: (i, 0))
    col = pl.BlockSpec((_BLK, 1), lambda i: (i, 0))
    vec = pl.BlockSpec((1, H), lambda i: (0, 0))
    f32 = jnp.float32
    return pl.pallas_call(
        _fin_body,
        grid=grid,
        in_specs=[row, col, vec, row, col, vec],
        out_specs=[row, row],
        out_shape=[jax.ShapeDtypeStruct((N, H), f32),
                   jax.ShapeDtypeStruct((N, H), f32)],
    )(num_i, den_i.reshape(N, 1), b_i.reshape(1, H),
      num_u, den_u.reshape(N, 1), b_u.reshape(1, H))


def _conv_edges(tid, h_hbm, asrc_hbm, adst_hbm, src_hbm, dst_hbm,
                numer_out, denom_out,
                a_src_v, a_dst_v, src_v, dst_v, w_v, rows_v,
                num_acc, den_acc, sem):
    pltpu.sync_copy(asrc_hbm, a_src_v)
    pltpu.sync_copy(adst_hbm, a_dst_v)

    def zrow(i, _):
        w_v[pl.ds(i * LANES, LANES)] = jnp.zeros((LANES,), jnp.float32)
        return 0
    lax.fori_loop(0, EC // LANES, zrow, 0)

    def zrows(i, _):
        r = i // 8
        c = lax.rem(i, 8) * LANES
        rows_v[r, pl.ds(c, LANES)] = jnp.zeros((LANES,), jnp.float32)
        return 0
    lax.fori_loop(0, EC * 8, zrows, 0)

    @pl.when(tid < N // ROWB)
    def _zero_acc():
        for k in range(ROWB // DC):
            off = tid * ROWB + k * DC
            pltpu.sync_copy(rows_v.at[pl.ds(0, DC)],
                            num_acc.at[pl.ds(off, DC)])
            pltpu.sync_copy(w_v.at[pl.ds(0, DC)],
                            den_acc.at[pl.ds(off, DC)])

    plsc.subcore_barrier()

    def chunk(g, _):
        base = tid * EPT + g * EC
        pltpu.sync_copy(src_hbm.at[pl.ds(base, EC)], src_v)
        pltpu.sync_copy(dst_hbm.at[pl.ds(base, EC)], dst_v)
        gcp = pltpu.async_copy(h_hbm.at[src_v], rows_v, sem)

        def wbody(j, _):
            s16 = src_v[pl.ds(j * LANES, LANES)]
            d16 = dst_v[pl.ds(j * LANES, LANES)]
            e = (plsc.load_gather(a_src_v, [s16]) +
                 plsc.load_gather(a_dst_v, [d16]))
            e = jnp.where(e > 0.0, e, e * 0.2)
            w_v[pl.ds(j * LANES, LANES)] = jnp.exp(e)
            return 0
        lax.fori_loop(0, EC // LANES, wbody, 0)

        pltpu.sync_copy(w_v, den_acc.at[dst_v], add=True)

        gcp.wait()

        def sbody(i, _):
            wv = plsc.load_gather(w_v, [jnp.full((LANES,), i, jnp.int32)])
            for j in range(H // LANES):
                c = j * LANES
                rows_v[i, pl.ds(c, LANES)] = rows_v[i, pl.ds(c, LANES)] * wv
            return 0
        lax.fori_loop(0, EC, sbody, 0)

        pltpu.sync_copy(rows_v, num_acc.at[dst_v], add=True)
        return 0
    lax.fori_loop(0, NCHUNK, chunk, 0)

    plsc.subcore_barrier()

    @pl.when(tid < N // ROWB)
    def _drain():
        for k in range(ROWB // DC):
            off = tid * ROWB + k * DC
            pltpu.sync_copy(num_acc.at[pl.ds(off, DC)],
                            rows_v.at[pl.ds(0, DC)])
            pltpu.sync_copy(rows_v.at[pl.ds(0, DC)],
                            numer_out.at[pl.ds(off, DC)])
            pltpu.sync_copy(den_acc.at[pl.ds(off, DC)],
                            w_v.at[pl.ds(0, DC)])
            pltpu.sync_copy(w_v.at[pl.ds(0, DC)],
                            denom_out.at[pl.ds(off, DC)])


def _edge_kernel_body(ha_hbm, sa_a_hbm, da_a_hbm, src_a_hbm, dst_a_hbm,
                      hb_hbm, sa_b_hbm, da_b_hbm, src_b_hbm, dst_b_hbm,
                      num_a_out, den_a_out, num_b_out, den_b_out,
                      a_src_v, a_dst_v, src_v, dst_v, w_v, rows_v,
                      num_acc, den_acc, sem):
    cid = lax.axis_index("c")
    tid = lax.axis_index("s")

    @pl.when(cid == 0)
    def _():
        _conv_edges(tid, ha_hbm, sa_a_hbm, da_a_hbm, src_a_hbm, dst_a_hbm,
                    num_a_out, den_a_out,
                    a_src_v, a_dst_v, src_v, dst_v, w_v, rows_v,
                    num_acc, den_acc, sem)

    @pl.when(cid == 1)
    def _():
        _conv_edges(tid, hb_hbm, sa_b_hbm, da_b_hbm, src_b_hbm, dst_b_hbm,
                    num_b_out, den_b_out,
                    a_src_v, a_dst_v, src_v, dst_v, w_v, rows_v,
                    num_acc, den_acc, sem)


def _edge_pass(ha, sa_a, da_a, edge_a, hb, sa_b, da_b, edge_b):
    f32 = jnp.float32
    mesh = plsc.VectorSubcoreMesh(core_axis_name="c", subcore_axis_name="s")
    run = functools.partial(
        pl.kernel,
        out_type=[
            jax.ShapeDtypeStruct((N, H), f32),
            jax.ShapeDtypeStruct((N,), f32),
            jax.ShapeDtypeStruct((N, H), f32),
            jax.ShapeDtypeStruct((N,), f32),
        ],
        mesh=mesh,
        compiler_params=pltpu.CompilerParams(needs_layout_passes=False),
        scratch_types=[
            pltpu.VMEM((N,), f32),
            pltpu.VMEM((N,), f32),
            pltpu.VMEM((EC,), jnp.int32),
            pltpu.VMEM((EC,), jnp.int32),
            pltpu.VMEM((EC,), f32),
            pltpu.VMEM((EC, H), f32),
            pltpu.VMEM_SHARED((N, H), f32),
            pltpu.VMEM_SHARED((N,), f32),
            pltpu.SemaphoreType.DMA,
        ],
    )
    k = run(_edge_kernel_body)
    return k(ha, sa_a.reshape(N), da_a.reshape(N), edge_a[0], edge_a[1],
             hb, sa_b.reshape(N), da_b.reshape(N), edge_b[0], edge_b[1])


def kernel(x_user, x_item, edge_index_u2i, edge_index_i2u,
           W_0_u2i, as_0_u2i, ad_0_u2i, b_0_u2i,
           W_0_i2u, as_0_i2u, ad_0_i2u, b_0_i2u,
           W_1_u2i, as_1_u2i, ad_1_u2i, b_1_u2i,
           W_1_i2u, as_1_i2u, ad_1_i2u, b_1_i2u):
    xu, xi = x_user, x_item
    eu, ei = edge_index_u2i, edge_index_i2u
    params = [
        (W_0_u2i, as_0_u2i, ad_0_u2i, b_0_u2i,
         W_0_i2u, as_0_i2u, ad_0_i2u, b_0_i2u),
        (W_1_u2i, as_1_u2i, ad_1_u2i, b_1_u2i,
         W_1_i2u, as_1_i2u, ad_1_i2u, b_1_i2u),
    ]
    for (wa, asa, ada, ba, wb, asb, adb, bb) in params:
        ha, saa, daa, hb, sab, dab = _project(xu, xi, wa, asa, ada,
                                              wb, asb, adb)
        num_a, den_a, num_b, den_b = _edge_pass(ha, saa, daa, eu,
                                                hb, sab, dab, ei)
        xi, xu = _finalize(num_a, den_a, ba, num_b, den_b, bb)
    return xu, xi


# speedup vs baseline: 50.4221x; 2.3333x over previous
import functools

import jax
import jax.numpy as jnp
from jax import lax
from jax.experimental import pallas as pl
from jax.experimental.pallas import tpu as pltpu
from jax.experimental.pallas import tpu_sc as plsc

N = 10000
H = 128
E = 320000
NS = 16
NC = 2
LANES = 16
EPT = E // NS
EC = 80
NCHUNK = EPT // EC
ROWB = 1000
DC = 40
EPS = 1e-16


_BLK = 1000


def _proj_body(xu_ref, xi_ref, wa_ref, asa_ref, ada_ref, wb_ref, asb_ref,
               adb_ref, ha_ref, saa_ref, daa_ref, hb_ref, sab_ref, dab_ref):
    xu = xu_ref[...]
    xi = xi_ref[...]
    wa = wa_ref[...]
    wb = wb_ref[...]
    ha = jnp.dot(xu, wa, preferred_element_type=jnp.float32)
    hb = jnp.dot(xi, wb, preferred_element_type=jnp.float32)
    ha_ref[...] = ha
    hb_ref[...] = hb
    saa_ref[...] = jnp.sum(ha * asa_ref[...], axis=1, keepdims=True)
    sab_ref[...] = jnp.sum(hb * asb_ref[...], axis=1, keepdims=True)
    va = jnp.sum(wa * ada_ref[...], axis=1, keepdims=True)
    vb = jnp.sum(wb * adb_ref[...], axis=1, keepdims=True)
    daa_ref[...] = jnp.dot(xi, va, preferred_element_type=jnp.float32)
    dab_ref[...] = jnp.dot(xu, vb, preferred_element_type=jnp.float32)


def _project(xu, xi, wa, asa, ada, wb, asb, adb):
    grid = (N // _BLK,)
    row = pl.BlockSpec((_BLK, H), lambda i: (i, 0))
    full = pl.BlockSpec((H, H), lambda i: (0, 0))
    vec = pl.BlockSpec((1, H), lambda i: (0, 0))
    col = pl.BlockSpec((_BLK, 1), lambda i: (i, 0))
    f32 = jnp.float32
    return pl.pallas_call(
        _proj_body,
        grid=grid,
        in_specs=[row, row, full, vec, vec, full, vec, vec],
        out_specs=[row, col, col, row, col, col],
        out_shape=[
            jax.ShapeDtypeStruct((N, H), f32),
            jax.ShapeDtypeStruct((N, 1), f32),
            jax.ShapeDtypeStruct((N, 1), f32),
            jax.ShapeDtypeStruct((N, H), f32),
            jax.ShapeDtypeStruct((N, 1), f32),
            jax.ShapeDtypeStruct((N, 1), f32),
        ],
    )(xu, xi, wa, asa.reshape(1, H), ada.reshape(1, H),
      wb, asb.reshape(1, H), adb.reshape(1, H))


def _fin_body(ni_ref, di_ref, bi_ref, nu_ref, du_ref, bu_ref,
              xi_ref, xu_ref):
    xi_ref[...] = jnp.maximum(
        ni_ref[...] / (di_ref[...] + EPS) + bi_ref[...], 0.0)
    xu_ref[...] = jnp.maximum(
        nu_ref[...] / (du_ref[...] + EPS) + bu_ref[...], 0.0)


def _finalize(num_i, den_i, b_i, num_u, den_u, b_u):
    grid = (N // _BLK,)
    row = pl.BlockSpec((_BLK, H), lambda i: (i, 0))
    col = pl.BlockSpec((_BLK, 1), lambda i: (i, 0))
    vec = pl.BlockSpec((1, H), lambda i: (0, 0))
    f32 = jnp.float32
    return pl.pallas_call(
        _fin_body,
        grid=grid,
        in_specs=[row, col, vec, row, col, vec],
        out_specs=[row, row],
        out_shape=[jax.ShapeDtypeStruct((N, H), f32),
                   jax.ShapeDtypeStruct((N, H), f32)],
    )(num_i, den_i.reshape(N, 1), b_i.reshape(1, H),
      num_u, den_u.reshape(N, 1), b_u.reshape(1, H))


_DEPTH = 4


def _conv_edges(tid, h_hbm, asrc_hbm, adst_hbm, src_hbm, dst_hbm,
                numer_out, denom_out, bufs, den_v, num_acc, den_acc, sems):
    srcs, dsts, ws, asbs, adbs, rows = zip(*bufs)
    isems, gsems, asems, ssems = zip(*sems)
    rows0 = rows[0]

    def zrows(i, _):
        r = i // 8
        c = lax.rem(i, 8) * LANES
        rows0[r, pl.ds(c, LANES)] = jnp.zeros((LANES,), jnp.float32)
        return 0
    lax.fori_loop(0, EC * 8, zrows, 0)

    def zden(i, _):
        den_v[pl.ds(i * LANES, LANES)] = jnp.zeros((LANES,), jnp.float32)
        return 0
    lax.fori_loop(0, EC // LANES, zden, 0)

    @pl.when(tid < N // ROWB)
    def _zero_acc():
        for k in range(ROWB // DC):
            off = tid * ROWB + k * DC
            pltpu.sync_copy(rows0.at[pl.ds(0, DC)],
                            num_acc.at[pl.ds(off, DC)])
            pltpu.sync_copy(den_v.at[pl.ds(0, DC)],
                            den_acc.at[pl.ds(off, DC)])

    plsc.subcore_barrier()

    def wcompute(q):
        def wbody(j, _):
            s = pl.ds(j * LANES, LANES)
            e = asbs[q][s] + adbs[q][s]
            e = jnp.where(e > 0.0, e, e * 0.2)
            ws[q][s] = jnp.exp(e)
            return 0
        lax.fori_loop(0, EC // LANES, wbody, 0)

    def scale(q):
        def sbody(i, _):
            wv = plsc.load_gather(ws[q],
                                  [jnp.full((LANES,), i, jnp.int32)])
            for j in range(H // LANES):
                c = j * LANES
                rows[q][i, pl.ds(c, LANES)] = rows[q][i, pl.ds(c, LANES)] * wv
            return 0
        lax.fori_loop(0, EC, sbody, 0)

    def fire_fetch(g, q):
        pltpu.async_copy(h_hbm.at[srcs[q]], rows[q], gsems[q])
        pltpu.async_copy(asrc_hbm.at[srcs[q]], asbs[q], asems[q])
        pltpu.async_copy(adst_hbm.at[dsts[q]], adbs[q], asems[q])

    def consume(g, q):
        pltpu.make_async_copy(h_hbm.at[srcs[q]], rows[q], gsems[q]).wait()
        pltpu.make_async_copy(asrc_hbm.at[srcs[q]], asbs[q], asems[q]).wait()
        pltpu.make_async_copy(adst_hbm.at[dsts[q]], adbs[q], asems[q]).wait()
        wcompute(q)
        pltpu.async_copy(ws[q], den_acc.at[dsts[q]], ssems[q], add=True)
        scale(q)
        pltpu.async_copy(rows[q], num_acc.at[dsts[q]], ssems[q], add=True)

    def wait_scatters(q):
        pltpu.make_async_copy(ws[q], den_acc.at[dsts[q]], ssems[q]).wait()
        pltpu.make_async_copy(rows[q], num_acc.at[dsts[q]], ssems[q]).wait()

    def slot(g, q, qn):
        wait_scatters(qn)
        pltpu.async_copy(src_hbm.at[tid, g + 2], srcs[qn], isems[qn])
        pltpu.async_copy(dst_hbm.at[tid, g + 2], dsts[qn], isems[qn])
        consume(g, q)
        pltpu.make_async_copy(src_hbm.at[tid, g + 2], srcs[qn],
                              isems[qn]).wait()
        pltpu.make_async_copy(dst_hbm.at[tid, g + 2], dsts[qn],
                              isems[qn]).wait()
        fire_fetch(g + 2, qn)

    for g in range(_DEPTH):
        pltpu.sync_copy(src_hbm.at[tid, g], srcs[g])
        pltpu.sync_copy(dst_hbm.at[tid, g], dsts[g])
    fire_fetch(0, 0)
    fire_fetch(1, 1)
    consume(0, 0)
    fire_fetch(2, 2)
    consume(1, 1)
    fire_fetch(3, 3)

    def quad(i, _):
        g0 = 4 * i + 2
        slot(g0 + 0, 2, 0)
        slot(g0 + 1, 3, 1)
        slot(g0 + 2, 0, 2)
        slot(g0 + 3, 1, 3)
        return 0
    lax.fori_loop(0, (NCHUNK - 6) // _DEPTH, quad, 0)

    slot(NCHUNK - 4, 2, 0)
    slot(NCHUNK - 3, 3, 1)
    consume(NCHUNK - 2, 0)
    consume(NCHUNK - 1, 1)
    for q in range(_DEPTH):
        wait_scatters(q)

    plsc.subcore_barrier()

    @pl.when(tid < N // ROWB)
    def _drain():
        for k in range(ROWB // DC):
            off = tid * ROWB + k * DC
            pltpu.sync_copy(num_acc.at[pl.ds(off, DC)],
                            rows0.at[pl.ds(0, DC)])
            pltpu.sync_copy(rows0.at[pl.ds(0, DC)],
                            numer_out.at[pl.ds(off, DC)])
            pltpu.sync_copy(den_acc.at[pl.ds(off, DC)],
                            den_v.at[pl.ds(0, DC)])
            pltpu.sync_copy(den_v.at[pl.ds(0, DC)],
                            denom_out.at[pl.ds(off, DC)])


def _edge_kernel_body(ha_hbm, sa_a_hbm, da_a_hbm, src_a_hbm, dst_a_hbm,
                      hb_hbm, sa_b_hbm, da_b_hbm, src_b_hbm, dst_b_hbm,
                      num_a_out, den_a_out, num_b_out, den_b_out,
                      *scratch):
    bufs = [scratch[6 * q:6 * q + 6] for q in range(_DEPTH)]
    den_v = scratch[6 * _DEPTH]
    num_acc = scratch[6 * _DEPTH + 1]
    den_acc = scratch[6 * _DEPTH + 2]
    s0 = 6 * _DEPTH + 3
    sems = [scratch[s0 + 4 * q:s0 + 4 * q + 4] for q in range(_DEPTH)]

    cid = lax.axis_index("c")
    tid = lax.axis_index("s")

    @pl.when(cid == 0)
    def _():
        _conv_edges(tid, ha_hbm, sa_a_hbm, da_a_hbm, src_a_hbm, dst_a_hbm,
                    num_a_out, den_a_out, bufs, den_v, num_acc, den_acc,
                    sems)

    @pl.when(cid == 1)
    def _():
        _conv_edges(tid, hb_hbm, sa_b_hbm, da_b_hbm, src_b_hbm, dst_b_hbm,
                    num_b_out, den_b_out, bufs, den_v, num_acc, den_acc,
                    sems)


def _edge_pass(ha, sa_a, da_a, edge_a, hb, sa_b, da_b, edge_b):
    f32 = jnp.float32
    i32 = jnp.int32
    mesh = plsc.VectorSubcoreMesh(core_axis_name="c", subcore_axis_name="s")
    set_scratch = [
        pltpu.VMEM((EC,), i32),
        pltpu.VMEM((EC,), i32),
        pltpu.VMEM((EC,), f32),
        pltpu.VMEM((EC,), f32),
        pltpu.VMEM((EC,), f32),
        pltpu.VMEM((EC, H), f32),
    ]
    run = functools.partial(
        pl.kernel,
        out_type=[
            jax.ShapeDtypeStruct((N, H), f32),
            jax.ShapeDtypeStruct((N,), f32),
            jax.ShapeDtypeStruct((N, H), f32),
            jax.ShapeDtypeStruct((N,), f32),
        ],
        mesh=mesh,
        compiler_params=pltpu.CompilerParams(needs_layout_passes=False),
        scratch_types=(
            set_scratch * _DEPTH
            + [
                pltpu.VMEM((EC,), f32),
                pltpu.VMEM_SHARED((N, H), f32),
                pltpu.VMEM_SHARED((N,), f32),
            ]
            + [pltpu.SemaphoreType.DMA] * (4 * _DEPTH)
        ),
    )
    k = run(_edge_kernel_body)
    ea0 = edge_a[0].reshape(NS, NCHUNK, EC)
    ea1 = edge_a[1].reshape(NS, NCHUNK, EC)
    eb0 = edge_b[0].reshape(NS, NCHUNK, EC)
    eb1 = edge_b[1].reshape(NS, NCHUNK, EC)
    return k(ha, sa_a.reshape(N), da_a.reshape(N), ea0, ea1,
             hb, sa_b.reshape(N), da_b.reshape(N), eb0, eb1)


def kernel(x_user, x_item, edge_index_u2i, edge_index_i2u,
           W_0_u2i, as_0_u2i, ad_0_u2i, b_0_u2i,
           W_0_i2u, as_0_i2u, ad_0_i2u, b_0_i2u,
           W_1_u2i, as_1_u2i, ad_1_u2i, b_1_u2i,
           W_1_i2u, as_1_i2u, ad_1_i2u, b_1_i2u):
    xu, xi = x_user, x_item
    eu, ei = edge_index_u2i, edge_index_i2u
    params = [
        (W_0_u2i, as_0_u2i, ad_0_u2i, b_0_u2i,
         W_0_i2u, as_0_i2u, ad_0_i2u, b_0_i2u),
        (W_1_u2i, as_1_u2i, ad_1_u2i, b_1_u2i,
         W_1_i2u, as_1_i2u, ad_1_i2u, b_1_i2u),
    ]
    for (wa, asa, ada, ba, wb, asb, adb, bb) in params:
        ha, saa, daa, hb, sab, dab = _project(xu, xi, wa, asa, ada,
                                              wb, asb, adb)
        num_a, den_a, num_b, den_b = _edge_pass(ha, saa, daa, eu,
                                                hb, sab, dab, ei)
        xi, xu = _finalize(num_a, den_a, ba, num_b, den_b, bb)
    return xu, xi


# speedup vs baseline: 52.3405x; 1.0380x over previous
import functools

import jax
import jax.numpy as jnp
from jax import lax
from jax.experimental import pallas as pl
from jax.experimental.pallas import tpu as pltpu
from jax.experimental.pallas import tpu_sc as plsc

N = 10000
H = 128
E = 320000
NS = 16
NC = 2
LANES = 16
EPT = E // NS
EC = 80
NCHUNK = EPT // EC
ROWB = 1000
DC = 40
ZB = 1040
EPS = 1e-16


_BLK = 1000


def _proj_body(xu_ref, xi_ref, wa_ref, asa_ref, ada_ref, wb_ref, asb_ref,
               adb_ref, ha_ref, saa_ref, daa_ref, hb_ref, sab_ref, dab_ref):
    xu = xu_ref[...]
    xi = xi_ref[...]
    wa = wa_ref[...]
    wb = wb_ref[...]
    ha = jnp.dot(xu, wa, preferred_element_type=jnp.float32)
    hb = jnp.dot(xi, wb, preferred_element_type=jnp.float32)
    ha_ref[...] = ha
    hb_ref[...] = hb
    saa_ref[...] = jnp.sum(ha * asa_ref[...], axis=1, keepdims=True)
    sab_ref[...] = jnp.sum(hb * asb_ref[...], axis=1, keepdims=True)
    va = jnp.sum(wa * ada_ref[...], axis=1, keepdims=True)
    vb = jnp.sum(wb * adb_ref[...], axis=1, keepdims=True)
    daa_ref[...] = jnp.dot(xi, va, preferred_element_type=jnp.float32)
    dab_ref[...] = jnp.dot(xu, vb, preferred_element_type=jnp.float32)


def _project(xu, xi, wa, asa, ada, wb, asb, adb):
    grid = (N // _BLK,)
    row = pl.BlockSpec((_BLK, H), lambda i: (i, 0))
    full = pl.BlockSpec((H, H), lambda i: (0, 0))
    vec = pl.BlockSpec((1, H), lambda i: (0, 0))
    col = pl.BlockSpec((_BLK, 1), lambda i: (i, 0))
    f32 = jnp.float32
    return pl.pallas_call(
        _proj_body,
        grid=grid,
        in_specs=[row, row, full, vec, vec, full, vec, vec],
        out_specs=[row, col, col, row, col, col],
        out_shape=[
            jax.ShapeDtypeStruct((N, H), f32),
            jax.ShapeDtypeStruct((N, 1), f32),
            jax.ShapeDtypeStruct((N, 1), f32),
            jax.ShapeDtypeStruct((N, H), f32),
            jax.ShapeDtypeStruct((N, 1), f32),
            jax.ShapeDtypeStruct((N, 1), f32),
        ],
    )(xu, xi, wa, asa.reshape(1, H), ada.reshape(1, H),
      wb, asb.reshape(1, H), adb.reshape(1, H))


def _fin_body(ni_ref, di_ref, bi_ref, nu_ref, du_ref, bu_ref,
              xi_ref, xu_ref):
    xi_ref[...] = jnp.maximum(
        ni_ref[...] / (di_ref[...] + EPS) + bi_ref[...], 0.0)
    xu_ref[...] = jnp.maximum(
        nu_ref[...] / (du_ref[...] + EPS) + bu_ref[...], 0.0)


def _finalize(num_i, den_i, b_i, num_u, den_u, b_u):
    grid = (N // _BLK,)
    row = pl.BlockSpec((_BLK, H), lambda i: (i, 0))
    col = pl.BlockSpec((_BLK, 1), lambda i: (i, 0))
    vec = pl.BlockSpec((1, H), lambda i: (0, 0))
    f32 = jnp.float32
    return pl.pallas_call(
        _fin_body,
        grid=grid,
        in_specs=[row, col, vec, row, col, vec],
        out_specs=[row, row],
        out_shape=[jax.ShapeDtypeStruct((N, H), f32),
                   jax.ShapeDtypeStruct((N, H), f32)],
    )(num_i, den_i.reshape(N, 1), b_i.reshape(1, H),
      num_u, den_u.reshape(N, 1), b_u.reshape(1, H))


_DEPTH = 4


def _conv_edges(tid, h_hbm, asrc_hbm, adst_hbm, src_hbm, dst_hbm,
                numer_out, denom_out, bufs, den_v, num_acc, den_acc, sems):
    srcs, dsts, ws, asbs, adbs, rows = zip(*bufs)
    isems, gsems, asems, ssems = zip(*sems)
    rows0 = rows[0]

    def zrows(i, _):
        r = i // 8
        c = lax.rem(i, 8) * LANES
        rows0[r, pl.ds(c, LANES)] = jnp.zeros((LANES,), jnp.float32)
        return 0
    lax.fori_loop(0, EC * 8, zrows, 0)

    def zden(i, _):
        den_v[pl.ds(i * LANES, LANES)] = jnp.zeros((LANES,), jnp.float32)
        return 0
    lax.fori_loop(0, ZB // LANES, zden, 0)

    @pl.when(tid < N // ROWB)
    def _zero_acc():
        off = tid * ROWB
        for k in range(ROWB // EC):
            pltpu.async_copy(rows0, num_acc.at[pl.ds(off + k * EC, EC)],
                             isems[k % _DEPTH])
        pltpu.sync_copy(rows0.at[pl.ds(0, DC)],
                        num_acc.at[pl.ds(off + (ROWB // EC) * EC, DC)])
        pltpu.sync_copy(den_v.at[pl.ds(0, ROWB)],
                        den_acc.at[pl.ds(off, ROWB)])
        for k in range(ROWB // EC):
            pltpu.make_async_copy(rows0,
                                  num_acc.at[pl.ds(off + k * EC, EC)],
                                  isems[k % _DEPTH]).wait()

    plsc.subcore_barrier()

    def wcompute(q):
        def wbody(j, _):
            s = pl.ds(j * LANES, LANES)
            e = asbs[q][s] + adbs[q][s]
            e = jnp.where(e > 0.0, e, e * 0.2)
            ws[q][s] = jnp.exp(e)
            return 0
        lax.fori_loop(0, EC // LANES, wbody, 0)

    def scale(q):
        def sbody(i, _):
            wv = plsc.load_gather(ws[q],
                                  [jnp.full((LANES,), i, jnp.int32)])
            for j in range(H // LANES):
                c = j * LANES
                rows[q][i, pl.ds(c, LANES)] = rows[q][i, pl.ds(c, LANES)] * wv
            return 0
        lax.fori_loop(0, EC, sbody, 0)

    def fire_fetch(g, q):
        pltpu.async_copy(h_hbm.at[srcs[q]], rows[q], gsems[q])
        pltpu.async_copy(asrc_hbm.at[srcs[q]], asbs[q], asems[q])
        pltpu.async_copy(adst_hbm.at[dsts[q]], adbs[q], asems[q])

    def consume(g, q):
        pltpu.make_async_copy(h_hbm.at[srcs[q]], rows[q], gsems[q]).wait()
        pltpu.make_async_copy(asrc_hbm.at[srcs[q]], asbs[q], asems[q]).wait()
        pltpu.make_async_copy(adst_hbm.at[dsts[q]], adbs[q], asems[q]).wait()
        wcompute(q)
        pltpu.async_copy(ws[q], den_acc.at[dsts[q]], ssems[q], add=True)
        scale(q)
        pltpu.async_copy(rows[q], num_acc.at[dsts[q]], ssems[q], add=True)

    def wait_scatters(q):
        pltpu.make_async_copy(ws[q], den_acc.at[dsts[q]], ssems[q]).wait()
        pltpu.make_async_copy(rows[q], num_acc.at[dsts[q]], ssems[q]).wait()

    def slot(g, q, qn):
        wait_scatters(qn)
        pltpu.async_copy(src_hbm.at[tid, g + 2], srcs[qn], isems[qn])
        pltpu.async_copy(dst_hbm.at[tid, g + 2], dsts[qn], isems[qn])
        consume(g, q)
        pltpu.make_async_copy(src_hbm.at[tid, g + 2], srcs[qn],
                              isems[qn]).wait()
        pltpu.make_async_copy(dst_hbm.at[tid, g + 2], dsts[qn],
                              isems[qn]).wait()
        fire_fetch(g + 2, qn)

    for g in range(_DEPTH):
        pltpu.sync_copy(src_hbm.at[tid, g], srcs[g])
        pltpu.sync_copy(dst_hbm.at[tid, g], dsts[g])
    fire_fetch(0, 0)
    fire_fetch(1, 1)
    consume(0, 0)
    fire_fetch(2, 2)
    consume(1, 1)
    fire_fetch(3, 3)

    def quad(i, _):
        g0 = 4 * i + 2
        slot(g0 + 0, 2, 0)
        slot(g0 + 1, 3, 1)
        slot(g0 + 2, 0, 2)
        slot(g0 + 3, 1, 3)
        return 0
    lax.fori_loop(0, (NCHUNK - 6) // _DEPTH, quad, 0)

    slot(NCHUNK - 4, 2, 0)
    slot(NCHUNK - 3, 3, 1)
    consume(NCHUNK - 2, 0)
    consume(NCHUNK - 1, 1)
    for q in range(_DEPTH):
        wait_scatters(q)

    plsc.subcore_barrier()

    @pl.when(tid < N // ROWB)
    def _drain():
        off = tid * ROWB
        pltpu.sync_copy(num_acc.at[pl.ds(off, ROWB)],
                        numer_out.at[pl.ds(off, ROWB)])
        pltpu.sync_copy(den_acc.at[pl.ds(off, ROWB)],
                        den_v.at[pl.ds(0, ROWB)])
        pltpu.sync_copy(den_v.at[pl.ds(0, ROWB)],
                        denom_out.at[pl.ds(off, ROWB)])


def _edge_kernel_body(ha_hbm, sa_a_hbm, da_a_hbm, src_a_hbm, dst_a_hbm,
                      hb_hbm, sa_b_hbm, da_b_hbm, src_b_hbm, dst_b_hbm,
                      num_a_out, den_a_out, num_b_out, den_b_out,
                      *scratch):
    bufs = [scratch[6 * q:6 * q + 6] for q in range(_DEPTH)]
    den_v = scratch[6 * _DEPTH]
    num_acc = scratch[6 * _DEPTH + 1]
    den_acc = scratch[6 * _DEPTH + 2]
    s0 = 6 * _DEPTH + 3
    sems = [scratch[s0 + 4 * q:s0 + 4 * q + 4] for q in range(_DEPTH)]

    cid = lax.axis_index("c")
    tid = lax.axis_index("s")

    @pl.when(cid == 0)
    def _():
        _conv_edges(tid, ha_hbm, sa_a_hbm, da_a_hbm, src_a_hbm, dst_a_hbm,
                    num_a_out, den_a_out, bufs, den_v, num_acc, den_acc,
                    sems)

    @pl.when(cid == 1)
    def _():
        _conv_edges(tid, hb_hbm, sa_b_hbm, da_b_hbm, src_b_hbm, dst_b_hbm,
                    num_b_out, den_b_out, bufs, den_v, num_acc, den_acc,
                    sems)


def _edge_pass(ha, sa_a, da_a, edge_a, hb, sa_b, da_b, edge_b):
    f32 = jnp.float32
    i32 = jnp.int32
    mesh = plsc.VectorSubcoreMesh(core_axis_name="c", subcore_axis_name="s")
    set_scratch = [
        pltpu.VMEM((EC,), i32),
        pltpu.VMEM((EC,), i32),
        pltpu.VMEM((EC,), f32),
        pltpu.VMEM((EC,), f32),
        pltpu.VMEM((EC,), f32),
        pltpu.VMEM((EC, H), f32),
    ]
    run = functools.partial(
        pl.kernel,
        out_type=[
            jax.ShapeDtypeStruct((N, H), f32),
            jax.ShapeDtypeStruct((N,), f32),
            jax.ShapeDtypeStruct((N, H), f32),
            jax.ShapeDtypeStruct((N,), f32),
        ],
        mesh=mesh,
        compiler_params=pltpu.CompilerParams(needs_layout_passes=False),
        scratch_types=(
            set_scratch * _DEPTH
            + [
                pltpu.VMEM((ZB,), f32),
                pltpu.VMEM_SHARED((N, H), f32),
                pltpu.VMEM_SHARED((N,), f32),
            ]
            + [pltpu.SemaphoreType.DMA] * (4 * _DEPTH)
        ),
    )
    k = run(_edge_kernel_body)
    ea0 = edge_a[0].reshape(NS, NCHUNK, EC)
    ea1 = edge_a[1].reshape(NS, NCHUNK, EC)
    eb0 = edge_b[0].reshape(NS, NCHUNK, EC)
    eb1 = edge_b[1].reshape(NS, NCHUNK, EC)
    return k(ha, sa_a.reshape(N), da_a.reshape(N), ea0, ea1,
             hb, sa_b.reshape(N), da_b.reshape(N), eb0, eb1)


def kernel(x_user, x_item, edge_index_u2i, edge_index_i2u,
           W_0_u2i, as_0_u2i, ad_0_u2i, b_0_u2i,
           W_0_i2u, as_0_i2u, ad_0_i2u, b_0_i2u,
           W_1_u2i, as_1_u2i, ad_1_u2i, b_1_u2i,
           W_1_i2u, as_1_i2u, ad_1_i2u, b_1_i2u):
    xu, xi = x_user, x_item
    eu, ei = edge_index_u2i, edge_index_i2u
    params = [
        (W_0_u2i, as_0_u2i, ad_0_u2i, b_0_u2i,
         W_0_i2u, as_0_i2u, ad_0_i2u, b_0_i2u),
        (W_1_u2i, as_1_u2i, ad_1_u2i, b_1_u2i,
         W_1_i2u, as_1_i2u, ad_1_i2u, b_1_i2u),
    ]
    for (wa, asa, ada, ba, wb, asb, adb, bb) in params:
        ha, saa, daa, hb, sab, dab = _project(xu, xi, wa, asa, ada,
                                              wb, asb, adb)
        num_a, den_a, num_b, den_b = _edge_pass(ha, saa, daa, eu,
                                                hb, sab, dab, ei)
        xi, xu = _finalize(num_a, den_a, ba, num_b, den_b, bb)
    return xu, xi


# speedup vs baseline: 57.2983x; 1.0947x over previous
import functools

import jax
import jax.numpy as jnp
from jax import lax
from jax.experimental import pallas as pl
from jax.experimental.pallas import tpu as pltpu
from jax.experimental.pallas import tpu_sc as plsc

N = 10000
H = 128
E = 320000
NS = 16
NC = 2
LANES = 16
EPT = E // NS
EC = 80
NCHUNK = EPT // EC
ROWB = 1000
DC = 40
ZB = 1040
EPS = 1e-16


_BLK = 1000


def _proj_body(xu_ref, xi_ref, wa_ref, asa_ref, ada_ref, wb_ref, asb_ref,
               adb_ref, ha_ref, saa_ref, daa_ref, hb_ref, sab_ref, dab_ref):
    xu = xu_ref[...]
    xi = xi_ref[...]
    wa = wa_ref[...]
    wb = wb_ref[...]
    ha = jnp.dot(xu, wa, preferred_element_type=jnp.float32)
    hb = jnp.dot(xi, wb, preferred_element_type=jnp.float32)
    ha_ref[...] = ha
    hb_ref[...] = hb
    saa_ref[...] = jnp.sum(ha * asa_ref[...], axis=1, keepdims=True)
    sab_ref[...] = jnp.sum(hb * asb_ref[...], axis=1, keepdims=True)
    va = jnp.sum(wa * ada_ref[...], axis=1, keepdims=True)
    vb = jnp.sum(wb * adb_ref[...], axis=1, keepdims=True)
    daa_ref[...] = jnp.dot(xi, va, preferred_element_type=jnp.float32)
    dab_ref[...] = jnp.dot(xu, vb, preferred_element_type=jnp.float32)


def _project(xu, xi, wa, asa, ada, wb, asb, adb):
    grid = (N // _BLK,)
    row = pl.BlockSpec((_BLK, H), lambda i: (i, 0))
    full = pl.BlockSpec((H, H), lambda i: (0, 0))
    vec = pl.BlockSpec((1, H), lambda i: (0, 0))
    col = pl.BlockSpec((_BLK, 1), lambda i: (i, 0))
    f32 = jnp.float32
    return pl.pallas_call(
        _proj_body,
        grid=grid,
        in_specs=[row, row, full, vec, vec, full, vec, vec],
        out_specs=[row, col, col, row, col, col],
        out_shape=[
            jax.ShapeDtypeStruct((N, H), f32),
            jax.ShapeDtypeStruct((N, 1), f32),
            jax.ShapeDtypeStruct((N, 1), f32),
            jax.ShapeDtypeStruct((N, H), f32),
            jax.ShapeDtypeStruct((N, 1), f32),
            jax.ShapeDtypeStruct((N, 1), f32),
        ],
    )(xu, xi, wa, asa.reshape(1, H), ada.reshape(1, H),
      wb, asb.reshape(1, H), adb.reshape(1, H))


def _fin_body(ni_ref, di_ref, bi_ref, nu_ref, du_ref, bu_ref,
              xi_ref, xu_ref):
    xi_ref[...] = jnp.maximum(
        ni_ref[...] / (di_ref[...] + EPS) + bi_ref[...], 0.0)
    xu_ref[...] = jnp.maximum(
        nu_ref[...] / (du_ref[...] + EPS) + bu_ref[...], 0.0)


def _finalize(num_i, den_i, b_i, num_u, den_u, b_u):
    grid = (N // _BLK,)
    row = pl.BlockSpec((_BLK, H), lambda i: (i, 0))
    col = pl.BlockSpec((_BLK, 1), lambda i: (i, 0))
    vec = pl.BlockSpec((1, H), lambda i: (0, 0))
    f32 = jnp.float32
    return pl.pallas_call(
        _fin_body,
        grid=grid,
        in_specs=[row, col, vec, row, col, vec],
        out_specs=[row, row],
        out_shape=[jax.ShapeDtypeStruct((N, H), f32),
                   jax.ShapeDtypeStruct((N, H), f32)],
    )(num_i, den_i.reshape(N, 1), b_i.reshape(1, H),
      num_u, den_u.reshape(N, 1), b_u.reshape(1, H))


_DEPTH = 4


def _conv_edges(tid, h_hbm, asrc_hbm, adst_hbm, src_hbm, dst_hbm,
                numer_out, denom_out, bufs, den_v, num_acc, den_acc, sems):
    srcs, dsts, ws, asbs, adbs, rows = zip(*bufs)
    isems, gsems, asems, ssems = zip(*sems)
    rows0 = rows[0]

    def zrows(i, _):
        r = i // 8
        c = lax.rem(i, 8) * LANES
        rows0[r, pl.ds(c, LANES)] = jnp.zeros((LANES,), jnp.float32)
        return 0
    lax.fori_loop(0, EC * 8, zrows, 0)

    def zden(i, _):
        den_v[pl.ds(i * LANES, LANES)] = jnp.zeros((LANES,), jnp.float32)
        return 0
    lax.fori_loop(0, ZB // LANES, zden, 0)

    @pl.when(tid < N // ROWB)
    def _zero_acc():
        off = tid * ROWB
        for k in range(ROWB // EC):
            pltpu.async_copy(rows0, num_acc.at[pl.ds(off + k * EC, EC)],
                             isems[k % _DEPTH])
        pltpu.sync_copy(rows0.at[pl.ds(0, DC)],
                        num_acc.at[pl.ds(off + (ROWB // EC) * EC, DC)])
        pltpu.sync_copy(den_v.at[pl.ds(0, ROWB)],
                        den_acc.at[pl.ds(off, ROWB)])
        for k in range(ROWB // EC):
            pltpu.make_async_copy(rows0,
                                  num_acc.at[pl.ds(off + k * EC, EC)],
                                  isems[k % _DEPTH]).wait()

    plsc.subcore_barrier()

    def wcompute(q):
        def wbody(j, _):
            s = pl.ds(j * LANES, LANES)
            e = asbs[q][s] + adbs[q][s]
            e = jnp.where(e > 0.0, e, e * 0.2)
            ws[q][s] = jnp.exp(e)
            return 0
        lax.fori_loop(0, EC // LANES, wbody, 0)

    def scale(q):
        def sblk(b, _):
            base = b * LANES
            w16 = ws[q][pl.ds(base, LANES)]
            for k in range(LANES):
                i = base + k
                wk = w16[k]
                for j in range(H // LANES):
                    c = j * LANES
                    rows[q][i, pl.ds(c, LANES)] = (
                        rows[q][i, pl.ds(c, LANES)] * wk)
            return 0
        lax.fori_loop(0, EC // LANES, sblk, 0)

    def fire_fetch(g, q):
        pltpu.async_copy(h_hbm.at[srcs[q]], rows[q], gsems[q])
        pltpu.async_copy(asrc_hbm.at[srcs[q]], asbs[q], asems[q])
        pltpu.async_copy(adst_hbm.at[dsts[q]], adbs[q], asems[q])

    def consume(g, q):
        pltpu.make_async_copy(h_hbm.at[srcs[q]], rows[q], gsems[q]).wait()
        pltpu.make_async_copy(asrc_hbm.at[srcs[q]], asbs[q], asems[q]).wait()
        pltpu.make_async_copy(adst_hbm.at[dsts[q]], adbs[q], asems[q]).wait()
        wcompute(q)
        pltpu.async_copy(ws[q], den_acc.at[dsts[q]], ssems[q], add=True)
        scale(q)
        pltpu.async_copy(rows[q], num_acc.at[dsts[q]], ssems[q], add=True)

    def wait_scatters(q):
        pltpu.make_async_copy(ws[q], den_acc.at[dsts[q]], ssems[q]).wait()
        pltpu.make_async_copy(rows[q], num_acc.at[dsts[q]], ssems[q]).wait()

    def slot(g, q, qn):
        wait_scatters(qn)
        pltpu.async_copy(src_hbm.at[tid, g + 2], srcs[qn], isems[qn])
        pltpu.async_copy(dst_hbm.at[tid, g + 2], dsts[qn], isems[qn])
        consume(g, q)
        pltpu.make_async_copy(src_hbm.at[tid, g + 2], srcs[qn],
                              isems[qn]).wait()
        pltpu.make_async_copy(dst_hbm.at[tid, g + 2], dsts[qn],
                              isems[qn]).wait()
        fire_fetch(g + 2, qn)

    for g in range(_DEPTH):
        pltpu.sync_copy(src_hbm.at[tid, g], srcs[g])
        pltpu.sync_copy(dst_hbm.at[tid, g], dsts[g])
    fire_fetch(0, 0)
    fire_fetch(1, 1)
    consume(0, 0)
    fire_fetch(2, 2)
    consume(1, 1)
    fire_fetch(3, 3)

    def quad(i, _):
        g0 = 4 * i + 2
        slot(g0 + 0, 2, 0)
        slot(g0 + 1, 3, 1)
        slot(g0 + 2, 0, 2)
        slot(g0 + 3, 1, 3)
        return 0
    lax.fori_loop(0, (NCHUNK - 6) // _DEPTH, quad, 0)

    slot(NCHUNK - 4, 2, 0)
    slot(NCHUNK - 3, 3, 1)
    consume(NCHUNK - 2, 0)
    consume(NCHUNK - 1, 1)
    for q in range(_DEPTH):
        wait_scatters(q)

    plsc.subcore_barrier()

    @pl.when(tid < N // ROWB)
    def _drain():
        off = tid * ROWB
        pltpu.sync_copy(num_acc.at[pl.ds(off, ROWB)],
                        numer_out.at[pl.ds(off, ROWB)])
        pltpu.sync_copy(den_acc.at[pl.ds(off, ROWB)],
                        den_v.at[pl.ds(0, ROWB)])
        pltpu.sync_copy(den_v.at[pl.ds(0, ROWB)],
                        denom_out.at[pl.ds(off, ROWB)])


def _edge_kernel_body(ha_hbm, sa_a_hbm, da_a_hbm, src_a_hbm, dst_a_hbm,
                      hb_hbm, sa_b_hbm, da_b_hbm, src_b_hbm, dst_b_hbm,
                      num_a_out, den_a_out, num_b_out, den_b_out,
                      *scratch):
    bufs = [scratch[6 * q:6 * q + 6] for q in range(_DEPTH)]
    den_v = scratch[6 * _DEPTH]
    num_acc = scratch[6 * _DEPTH + 1]
    den_acc = scratch[6 * _DEPTH + 2]
    s0 = 6 * _DEPTH + 3
    sems = [scratch[s0 + 4 * q:s0 + 4 * q + 4] for q in range(_DEPTH)]

    cid = lax.axis_index("c")
    tid = lax.axis_index("s")

    @pl.when(cid == 0)
    def _():
        _conv_edges(tid, ha_hbm, sa_a_hbm, da_a_hbm, src_a_hbm, dst_a_hbm,
                    num_a_out, den_a_out, bufs, den_v, num_acc, den_acc,
                    sems)

    @pl.when(cid == 1)
    def _():
        _conv_edges(tid, hb_hbm, sa_b_hbm, da_b_hbm, src_b_hbm, dst_b_hbm,
                    num_b_out, den_b_out, bufs, den_v, num_acc, den_acc,
                    sems)


def _edge_pass(ha, sa_a, da_a, edge_a, hb, sa_b, da_b, edge_b):
    f32 = jnp.float32
    i32 = jnp.int32
    mesh = plsc.VectorSubcoreMesh(core_axis_name="c", subcore_axis_name="s")
    set_scratch = [
        pltpu.VMEM((EC,), i32),
        pltpu.VMEM((EC,), i32),
        pltpu.VMEM((EC,), f32),
        pltpu.VMEM((EC,), f32),
        pltpu.VMEM((EC,), f32),
        pltpu.VMEM((EC, H), f32),
    ]
    run = functools.partial(
        pl.kernel,
        out_type=[
            jax.ShapeDtypeStruct((N, H), f32),
            jax.ShapeDtypeStruct((N,), f32),
            jax.ShapeDtypeStruct((N, H), f32),
            jax.ShapeDtypeStruct((N,), f32),
        ],
        mesh=mesh,
        compiler_params=pltpu.CompilerParams(needs_layout_passes=False),
        scratch_types=(
            set_scratch * _DEPTH
            + [
                pltpu.VMEM((ZB,), f32),
                pltpu.VMEM_SHARED((N, H), f32),
                pltpu.VMEM_SHARED((N,), f32),
            ]
            + [pltpu.SemaphoreType.DMA] * (4 * _DEPTH)
        ),
    )
    k = run(_edge_kernel_body)
    ea0 = edge_a[0].reshape(NS, NCHUNK, EC)
    ea1 = edge_a[1].reshape(NS, NCHUNK, EC)
    eb0 = edge_b[0].reshape(NS, NCHUNK, EC)
    eb1 = edge_b[1].reshape(NS, NCHUNK, EC)
    return k(ha, sa_a.reshape(N), da_a.reshape(N), ea0, ea1,
             hb, sa_b.reshape(N), da_b.reshape(N), eb0, eb1)


def kernel(x_user, x_item, edge_index_u2i, edge_index_i2u,
           W_0_u2i, as_0_u2i, ad_0_u2i, b_0_u2i,
           W_0_i2u, as_0_i2u, ad_0_i2u, b_0_i2u,
           W_1_u2i, as_1_u2i, ad_1_u2i, b_1_u2i,
           W_1_i2u, as_1_i2u, ad_1_i2u, b_1_i2u):
    xu, xi = x_user, x_item
    eu, ei = edge_index_u2i, edge_index_i2u
    params = [
        (W_0_u2i, as_0_u2i, ad_0_u2i, b_0_u2i,
         W_0_i2u, as_0_i2u, ad_0_i2u, b_0_i2u),
        (W_1_u2i, as_1_u2i, ad_1_u2i, b_1_u2i,
         W_1_i2u, as_1_i2u, ad_1_i2u, b_1_i2u),
    ]
    for (wa, asa, ada, ba, wb, asb, adb, bb) in params:
        ha, saa, daa, hb, sab, dab = _project(xu, xi, wa, asa, ada,
                                              wb, asb, adb)
        num_a, den_a, num_b, den_b = _edge_pass(ha, saa, daa, eu,
                                                hb, sab, dab, ei)
        xi, xu = _finalize(num_a, den_a, ba, num_b, den_b, bb)
    return xu, xi


# speedup vs baseline: 58.2472x; 1.0166x over previous
import functools

import jax
import jax.numpy as jnp
from jax import lax
from jax.experimental import pallas as pl
from jax.experimental.pallas import tpu as pltpu
from jax.experimental.pallas import tpu_sc as plsc

N = 10000
H = 128
E = 320000
NS = 16
NC = 2
LANES = 16
EPT = E // NS
EC = 80
NCHUNK = EPT // EC
ROWB = 1000
DC = 40
ZB = 1040
EPS = 1e-16


_BLK = 1000


def _proj_body(xu_ref, xi_ref, wa_ref, asa_ref, ada_ref, wb_ref, asb_ref,
               adb_ref, ha_ref, saa_ref, daa_ref, hb_ref, sab_ref, dab_ref):
    xu = xu_ref[...]
    xi = xi_ref[...]
    wa = wa_ref[...]
    wb = wb_ref[...]
    ha = jnp.dot(xu, wa, preferred_element_type=jnp.float32)
    hb = jnp.dot(xi, wb, preferred_element_type=jnp.float32)
    ha_ref[...] = ha
    hb_ref[...] = hb
    saa_ref[...] = jnp.sum(ha * asa_ref[...], axis=1, keepdims=True)
    sab_ref[...] = jnp.sum(hb * asb_ref[...], axis=1, keepdims=True)
    va = jnp.sum(wa * ada_ref[...], axis=1, keepdims=True)
    vb = jnp.sum(wb * adb_ref[...], axis=1, keepdims=True)
    daa_ref[...] = jnp.dot(xi, va, preferred_element_type=jnp.float32)
    dab_ref[...] = jnp.dot(xu, vb, preferred_element_type=jnp.float32)


def _project(xu, xi, wa, asa, ada, wb, asb, adb):
    grid = (N // _BLK,)
    row = pl.BlockSpec((_BLK, H), lambda i: (i, 0))
    full = pl.BlockSpec((H, H), lambda i: (0, 0))
    vec = pl.BlockSpec((1, H), lambda i: (0, 0))
    col = pl.BlockSpec((_BLK, 1), lambda i: (i, 0))
    f32 = jnp.float32
    return pl.pallas_call(
        _proj_body,
        grid=grid,
        in_specs=[row, row, full, vec, vec, full, vec, vec],
        out_specs=[row, col, col, row, col, col],
        out_shape=[
            jax.ShapeDtypeStruct((N, H), f32),
            jax.ShapeDtypeStruct((N, 1), f32),
            jax.ShapeDtypeStruct((N, 1), f32),
            jax.ShapeDtypeStruct((N, H), f32),
            jax.ShapeDtypeStruct((N, 1), f32),
            jax.ShapeDtypeStruct((N, 1), f32),
        ],
    )(xu, xi, wa, asa.reshape(1, H), ada.reshape(1, H),
      wb, asb.reshape(1, H), adb.reshape(1, H))


def _finproj_body(na_ref, da_ref, ba_ref, nb_ref, db_ref, bb_ref,
                  wa_ref, asa_ref, ada_ref, wb_ref, asb_ref, adb_ref,
                  ha_ref, saa_ref, daa_ref, hb_ref, sab_ref, dab_ref):
    xi = jnp.maximum(na_ref[...] / (da_ref[...] + EPS) + ba_ref[...], 0.0)
    xu = jnp.maximum(nb_ref[...] / (db_ref[...] + EPS) + bb_ref[...], 0.0)
    wa = wa_ref[...]
    wb = wb_ref[...]
    ha = jnp.dot(xu, wa, preferred_element_type=jnp.float32)
    hb = jnp.dot(xi, wb, preferred_element_type=jnp.float32)
    ha_ref[...] = ha
    hb_ref[...] = hb
    saa_ref[...] = jnp.sum(ha * asa_ref[...], axis=1, keepdims=True)
    sab_ref[...] = jnp.sum(hb * asb_ref[...], axis=1, keepdims=True)
    va = jnp.sum(wa * ada_ref[...], axis=1, keepdims=True)
    vb = jnp.sum(wb * adb_ref[...], axis=1, keepdims=True)
    daa_ref[...] = jnp.dot(xi, va, preferred_element_type=jnp.float32)
    dab_ref[...] = jnp.dot(xu, vb, preferred_element_type=jnp.float32)


def _finalize_project(num_a, den_a, ba, num_b, den_b, bb,
                      wa, asa, ada, wb, asb, adb):
    grid = (N // _BLK,)
    row = pl.BlockSpec((_BLK, H), lambda i: (i, 0))
    full = pl.BlockSpec((H, H), lambda i: (0, 0))
    vec = pl.BlockSpec((1, H), lambda i: (0, 0))
    col = pl.BlockSpec((_BLK, 1), lambda i: (i, 0))
    f32 = jnp.float32
    return pl.pallas_call(
        _finproj_body,
        grid=grid,
        in_specs=[row, col, vec, row, col, vec,
                  full, vec, vec, full, vec, vec],
        out_specs=[row, col, col, row, col, col],
        out_shape=[
            jax.ShapeDtypeStruct((N, H), f32),
            jax.ShapeDtypeStruct((N, 1), f32),
            jax.ShapeDtypeStruct((N, 1), f32),
            jax.ShapeDtypeStruct((N, H), f32),
            jax.ShapeDtypeStruct((N, 1), f32),
            jax.ShapeDtypeStruct((N, 1), f32),
        ],
    )(num_a, den_a.reshape(N, 1), ba.reshape(1, H),
      num_b, den_b.reshape(N, 1), bb.reshape(1, H),
      wa, asa.reshape(1, H), ada.reshape(1, H),
      wb, asb.reshape(1, H), adb.reshape(1, H))


def _fin_body(ni_ref, di_ref, bi_ref, nu_ref, du_ref, bu_ref,
              xi_ref, xu_ref):
    xi_ref[...] = jnp.maximum(
        ni_ref[...] / (di_ref[...] + EPS) + bi_ref[...], 0.0)
    xu_ref[...] = jnp.maximum(
        nu_ref[...] / (du_ref[...] + EPS) + bu_ref[...], 0.0)


def _finalize(num_i, den_i, b_i, num_u, den_u, b_u):
    grid = (N // _BLK,)
    row = pl.BlockSpec((_BLK, H), lambda i: (i, 0))
    col = pl.BlockSpec((_BLK, 1), lambda i: (i, 0))
    vec = pl.BlockSpec((1, H), lambda i: (0, 0))
    f32 = jnp.float32
    return pl.pallas_call(
        _fin_body,
        grid=grid,
        in_specs=[row, col, vec, row, col, vec],
        out_specs=[row, row],
        out_shape=[jax.ShapeDtypeStruct((N, H), f32),
                   jax.ShapeDtypeStruct((N, H), f32)],
    )(num_i, den_i.reshape(N, 1), b_i.reshape(1, H),
      num_u, den_u.reshape(N, 1), b_u.reshape(1, H))


_DEPTH = 4


def _conv_edges(tid, h_hbm, asrc_hbm, adst_hbm, src_hbm, dst_hbm,
                numer_out, denom_out, bufs, den_v, num_acc, den_acc, sems):
    srcs, dsts, ws, asbs, adbs, rows = zip(*bufs)
    isems, gsems, asems, ssems = zip(*sems)
    rows0 = rows[0]

    def zrows(i, _):
        r = i // 8
        c = lax.rem(i, 8) * LANES
        rows0[r, pl.ds(c, LANES)] = jnp.zeros((LANES,), jnp.float32)
        return 0
    lax.fori_loop(0, EC * 8, zrows, 0)

    def zden(i, _):
        den_v[pl.ds(i * LANES, LANES)] = jnp.zeros((LANES,), jnp.float32)
        return 0
    lax.fori_loop(0, ZB // LANES, zden, 0)

    @pl.when(tid < N // ROWB)
    def _zero_acc():
        off = tid * ROWB
        for k in range(ROWB // EC):
            pltpu.async_copy(rows0, num_acc.at[pl.ds(off + k * EC, EC)],
                             isems[k % _DEPTH])
        pltpu.sync_copy(rows0.at[pl.ds(0, DC)],
                        num_acc.at[pl.ds(off + (ROWB // EC) * EC, DC)])
        pltpu.sync_copy(den_v.at[pl.ds(0, ROWB)],
                        den_acc.at[pl.ds(off, ROWB)])
        for k in range(ROWB // EC):
            pltpu.make_async_copy(rows0,
                                  num_acc.at[pl.ds(off + k * EC, EC)],
                                  isems[k % _DEPTH]).wait()

    plsc.subcore_barrier()

    def wcompute(q):
        def wbody(j, _):
            s = pl.ds(j * LANES, LANES)
            e = asbs[q][s] + adbs[q][s]
            e = jnp.where(e > 0.0, e, e * 0.2)
            ws[q][s] = jnp.exp(e)
            return 0
        lax.fori_loop(0, EC // LANES, wbody, 0)

    def scale(q):
        def sblk(b, _):
            base = b * LANES
            w16 = ws[q][pl.ds(base, LANES)]
            for k in range(LANES):
                i = base + k
                wk = w16[k]
                for j in range(H // LANES):
                    c = j * LANES
                    rows[q][i, pl.ds(c, LANES)] = (
                        rows[q][i, pl.ds(c, LANES)] * wk)
            return 0
        lax.fori_loop(0, EC // LANES, sblk, 0)

    def fire_fetch(g, q):
        pltpu.async_copy(h_hbm.at[srcs[q]], rows[q], gsems[q])
        pltpu.async_copy(asrc_hbm.at[srcs[q]], asbs[q], asems[q])
        pltpu.async_copy(adst_hbm.at[dsts[q]], adbs[q], asems[q])

    def consume(g, q):
        pltpu.make_async_copy(h_hbm.at[srcs[q]], rows[q], gsems[q]).wait()
        pltpu.make_async_copy(asrc_hbm.at[srcs[q]], asbs[q], asems[q]).wait()
        pltpu.make_async_copy(adst_hbm.at[dsts[q]], adbs[q], asems[q]).wait()
        wcompute(q)
        pltpu.async_copy(ws[q], den_acc.at[dsts[q]], ssems[q], add=True)
        scale(q)
        pltpu.async_copy(rows[q], num_acc.at[dsts[q]], ssems[q], add=True)

    def wait_scatters(q):
        pltpu.make_async_copy(ws[q], den_acc.at[dsts[q]], ssems[q]).wait()
        pltpu.make_async_copy(rows[q], num_acc.at[dsts[q]], ssems[q]).wait()

    def slot(g, q, qn):
        wait_scatters(qn)
        pltpu.async_copy(src_hbm.at[tid, g + 2], srcs[qn], isems[qn])
        pltpu.async_copy(dst_hbm.at[tid, g + 2], dsts[qn], isems[qn])
        consume(g, q)
        pltpu.make_async_copy(src_hbm.at[tid, g + 2], srcs[qn],
                              isems[qn]).wait()
        pltpu.make_async_copy(dst_hbm.at[tid, g + 2], dsts[qn],
                              isems[qn]).wait()
        fire_fetch(g + 2, qn)

    for g in range(_DEPTH):
        pltpu.sync_copy(src_hbm.at[tid, g], srcs[g])
        pltpu.sync_copy(dst_hbm.at[tid, g], dsts[g])
    fire_fetch(0, 0)
    fire_fetch(1, 1)
    consume(0, 0)
    fire_fetch(2, 2)
    consume(1, 1)
    fire_fetch(3, 3)

    def quad(i, _):
        g0 = 4 * i + 2
        slot(g0 + 0, 2, 0)
        slot(g0 + 1, 3, 1)
        slot(g0 + 2, 0, 2)
        slot(g0 + 3, 1, 3)
        return 0
    lax.fori_loop(0, (NCHUNK - 6) // _DEPTH, quad, 0)

    slot(NCHUNK - 4, 2, 0)
    slot(NCHUNK - 3, 3, 1)
    consume(NCHUNK - 2, 0)
    consume(NCHUNK - 1, 1)
    for q in range(_DEPTH):
        wait_scatters(q)

    plsc.subcore_barrier()

    @pl.when(tid < N // ROWB)
    def _drain():
        off = tid * ROWB
        pltpu.sync_copy(num_acc.at[pl.ds(off, ROWB)],
                        numer_out.at[pl.ds(off, ROWB)])
        pltpu.sync_copy(den_acc.at[pl.ds(off, ROWB)],
                        den_v.at[pl.ds(0, ROWB)])
        pltpu.sync_copy(den_v.at[pl.ds(0, ROWB)],
                        denom_out.at[pl.ds(off, ROWB)])


def _edge_kernel_body(ha_hbm, sa_a_hbm, da_a_hbm, src_a_hbm, dst_a_hbm,
                      hb_hbm, sa_b_hbm, da_b_hbm, src_b_hbm, dst_b_hbm,
                      num_a_out, den_a_out, num_b_out, den_b_out,
                      *scratch):
    bufs = [scratch[6 * q:6 * q + 6] for q in range(_DEPTH)]
    den_v = scratch[6 * _DEPTH]
    num_acc = scratch[6 * _DEPTH + 1]
    den_acc = scratch[6 * _DEPTH + 2]
    s0 = 6 * _DEPTH + 3
    sems = [scratch[s0 + 4 * q:s0 + 4 * q + 4] for q in range(_DEPTH)]

    cid = lax.axis_index("c")
    tid = lax.axis_index("s")

    @pl.when(cid == 0)
    def _():
        _conv_edges(tid, ha_hbm, sa_a_hbm, da_a_hbm, src_a_hbm, dst_a_hbm,
                    num_a_out, den_a_out, bufs, den_v, num_acc, den_acc,
                    sems)

    @pl.when(cid == 1)
    def _():
        _conv_edges(tid, hb_hbm, sa_b_hbm, da_b_hbm, src_b_hbm, dst_b_hbm,
                    num_b_out, den_b_out, bufs, den_v, num_acc, den_acc,
                    sems)


def _edge_pass(ha, sa_a, da_a, edge_a, hb, sa_b, da_b, edge_b):
    f32 = jnp.float32
    i32 = jnp.int32
    mesh = plsc.VectorSubcoreMesh(core_axis_name="c", subcore_axis_name="s")
    set_scratch = [
        pltpu.VMEM((EC,), i32),
        pltpu.VMEM((EC,), i32),
        pltpu.VMEM((EC,), f32),
        pltpu.VMEM((EC,), f32),
        pltpu.VMEM((EC,), f32),
        pltpu.VMEM((EC, H), f32),
    ]
    run = functools.partial(
        pl.kernel,
        out_type=[
            jax.ShapeDtypeStruct((N, H), f32),
            jax.ShapeDtypeStruct((N,), f32),
            jax.ShapeDtypeStruct((N, H), f32),
            jax.ShapeDtypeStruct((N,), f32),
        ],
        mesh=mesh,
        compiler_params=pltpu.CompilerParams(needs_layout_passes=False),
        scratch_types=(
            set_scratch * _DEPTH
            + [
                pltpu.VMEM((ZB,), f32),
                pltpu.VMEM_SHARED((N, H), f32),
                pltpu.VMEM_SHARED((N,), f32),
            ]
            + [pltpu.SemaphoreType.DMA] * (4 * _DEPTH)
        ),
    )
    k = run(_edge_kernel_body)
    ea0 = edge_a[0].reshape(NS, NCHUNK, EC)
    ea1 = edge_a[1].reshape(NS, NCHUNK, EC)
    eb0 = edge_b[0].reshape(NS, NCHUNK, EC)
    eb1 = edge_b[1].reshape(NS, NCHUNK, EC)
    return k(ha, sa_a.reshape(N), da_a.reshape(N), ea0, ea1,
             hb, sa_b.reshape(N), da_b.reshape(N), eb0, eb1)


def kernel(x_user, x_item, edge_index_u2i, edge_index_i2u,
           W_0_u2i, as_0_u2i, ad_0_u2i, b_0_u2i,
           W_0_i2u, as_0_i2u, ad_0_i2u, b_0_i2u,
           W_1_u2i, as_1_u2i, ad_1_u2i, b_1_u2i,
           W_1_i2u, as_1_i2u, ad_1_i2u, b_1_i2u):
    xu, xi = x_user, x_item
    eu, ei = edge_index_u2i, edge_index_i2u
    ha, saa, daa, hb, sab, dab = _project(
        xu, xi, W_0_u2i, as_0_u2i, ad_0_u2i, W_0_i2u, as_0_i2u, ad_0_i2u)
    num_a, den_a, num_b, den_b = _edge_pass(ha, saa, daa, eu,
                                            hb, sab, dab, ei)
    ha, saa, daa, hb, sab, dab = _finalize_project(
        num_a, den_a, b_0_u2i, num_b, den_b, b_0_i2u,
        W_1_u2i, as_1_u2i, ad_1_u2i, W_1_i2u, as_1_i2u, ad_1_i2u)
    num_a, den_a, num_b, den_b = _edge_pass(ha, saa, daa, eu,
                                            hb, sab, dab, ei)
    xi, xu = _finalize(num_a, den_a, b_1_u2i, num_b, den_b, b_1_i2u)
    return xu, xi
